# baseline (speedup 1.0000x reference)
import dataclasses
import functools

import jax
import jax.numpy as jnp
from jax import lax
from jax.experimental import pallas as pl
from jax.experimental.pallas import tpu as pltpu
from jax.experimental.pallas import tpu_sc as plsc

_N = 10000
_NPAD = 10240
_E = 320000
_D = 128
_DP = 48
_EPS = 1e-5
_CHUNK = 125
_NBUF = 5
_EPT = _E // 16
_CPT = _EPT // _CHUNK
_EPW = _E // 32
_CPW = _EPW // _CHUNK
_HROWS = _NPAD // 16
_RPT = _NPAD // 16

_mesh = plsc.VectorSubcoreMesh(core_axis_name="c", subcore_axis_name="s")

_cp = pltpu.CompilerParams()
if "needs_layout_passes" in pltpu.CompilerParams.__dataclass_fields__:
    _cp = dataclasses.replace(_cp, needs_layout_passes=False)
_cp_flat = dataclasses.replace(pltpu.CompilerParams(),
                               use_tc_tiling_on_sc=False)
_cp_flat_nl = dataclasses.replace(_cp, use_tc_tiling_on_sc=False)


@functools.partial(
    pl.kernel,
    out_type=jax.ShapeDtypeStruct((2 * 2 * _HROWS, 16), jnp.float32),
    mesh=_mesh,
    compiler_params=_cp_flat_nl,
    scratch_types=[
        pltpu.VMEM((_EPW,), jnp.int32),
        pltpu.VMEM((_EPW,), jnp.int32),
        pltpu.VMEM((10, 128), jnp.int32),
        pltpu.VMEM((_HROWS, 16), jnp.float32),
        pltpu.VMEM((_HROWS, 16), jnp.float32),
        pltpu.VMEM_SHARED((2 * _HROWS, 16), jnp.float32),
    ],
)
def _deg_kernel(src_hbm, dst_hbm, idxrows_hbm, out_hbm,
                src_v, dst_v, idxr_v, hs_v, hd_v, spm):
    cid = lax.axis_index("c")
    sid = lax.axis_index("s")
    wid = cid * 16 + sid
    base = wid * _EPW
    pltpu.sync_copy(src_hbm.at[pl.ds(base, _EPW)], src_v)
    pltpu.sync_copy(dst_hbm.at[pl.ds(base, _EPW)], dst_v)
    pltpu.sync_copy(idxrows_hbm, idxr_v)

    zero16 = jnp.zeros((16,), jnp.float32)

    @pl.loop(0, _HROWS)
    def _(r):
        hs_v[r, :] = zero16
        hd_v[r, :] = zero16

    @pl.when(sid == 0)
    def _():
        pltpu.sync_copy(hs_v, spm.at[pl.ds(0, _HROWS)])
        pltpu.sync_copy(hd_v, spm.at[pl.ds(_HROWS, _HROWS)])

    plsc.subcore_barrier()

    ones = jnp.ones((16,), jnp.float32)

    @pl.loop(0, _EPW // 16)
    def _(i):
        s = src_v[pl.ds(i * 16, 16)]
        d = dst_v[pl.ds(i * 16, 16)]
        plsc.addupdate_scatter(hs_v, [s >> 4, s & 15], ones)
        plsc.addupdate_scatter(hd_v, [d >> 4, d & 15], ones)

    for j in range(5):
        pltpu.sync_copy(hs_v.at[pl.ds(j * 128, 128)],
                        spm.at[idxr_v.at[j]], add=True)
        pltpu.sync_copy(hd_v.at[pl.ds(j * 128, 128)],
                        spm.at[idxr_v.at[j + 5]], add=True)

    plsc.subcore_barrier()

    off = sid * (2 * _HROWS // 16)
    pltpu.sync_copy(spm.at[pl.ds(off, 2 * _HROWS // 16)],
                    out_hbm.at[pl.ds(cid * 2 * _HROWS + off, 2 * _HROWS // 16)])


def _zero_acc(bufs, acc, sid, width):
    zvec = jnp.zeros((16,), jnp.float32)

    @pl.loop(0, 80)
    def _(r):
        for v in range(width // 16):
            bufs[0, r, pl.ds(v * 16, 16)] = zvec

    zsrc = bufs.at[0].at[pl.ds(0, 80)]
    for j in range(_RPT // 80):
        pltpu.sync_copy(zsrc, acc.at[pl.ds(sid * _RPT + j * 80, 80)])


def _readout(bufs, acc, out_hbm, cid, sid):
    rbuf = bufs.at[0].at[pl.ds(0, 80)]
    for j in range(_RPT // 80):
        off = sid * _RPT + j * 80
        pltpu.sync_copy(acc.at[pl.ds(off, 80)], rbuf)
        pltpu.sync_copy(rbuf, out_hbm.at[pl.ds(cid * _NPAD + off, 80)])


def _edge_pipeline(gather_src, drain_src, didx_v, bufs, acc, gsems, ssems,
                   n_chunks):
    rounds = n_chunks // _NBUF
    sdrain = acc.at[pl.ds(0, _CHUNK)]

    def buf(b):
        return bufs.at[b].at[pl.ds(0, _CHUNK)]

    for b in range(3):
        pltpu.async_copy(gather_src(b), buf(b), gsems[b])

    @pl.loop(0, rounds)
    def _(t):
        for b in range(_NBUF):
            j = t * _NBUF + b
            b3 = (b + 3) % _NBUF
            pltpu.make_async_copy(drain_src, buf(b), gsems[b]).wait()
            pltpu.async_copy(buf(b), acc.at[didx_v.at[j]], ssems[b],
                             add=True)

            def _advance(j=j, b3=b3):
                pltpu.make_async_copy(buf(b3), sdrain, ssems[b3]).wait()
                pltpu.async_copy(gather_src(j + 3), buf(b3), gsems[b3])

            if b < 2:
                @pl.when(t > 0)
                def _():
                    _advance()

                @pl.when(t == 0)
                def _():
                    pltpu.async_copy(gather_src(j + 3), buf(b3), gsems[b3])
            else:
                @pl.when(t < rounds - 1)
                def _():
                    _advance()

    for jl in range(n_chunks - _NBUF, n_chunks):
        b = jl % _NBUF
        pltpu.make_async_copy(buf(b), sdrain, ssems[b]).wait()


@functools.partial(
    pl.kernel,
    out_type=jax.ShapeDtypeStruct((_NPAD, _D), jnp.float32),
    mesh=_mesh,
    compiler_params=_cp_flat,
    scratch_types=[
        pltpu.VMEM((_CPT, _CHUNK), jnp.int32),
        pltpu.VMEM((_CPT, _CHUNK), jnp.int32),
        pltpu.VMEM((_NBUF, _CHUNK, 64), jnp.float32),
        pltpu.VMEM_SHARED((_NPAD, 64), jnp.float32),
    ] + [pltpu.SemaphoreType.DMA] * (2 * _NBUF),
)
def _agg_half(hs_hbm, src_hbm, dst_hbm, out_hbm,
              sidx_v, didx_v, bufs, acc, *sems):
    cid = lax.axis_index("c")
    sid = lax.axis_index("s")
    gsems, ssems = sems[:_NBUF], sems[_NBUF:]
    pltpu.sync_copy(src_hbm.at[cid * 16 + sid], sidx_v)
    pltpu.sync_copy(dst_hbm.at[sid], didx_v)
    _zero_acc(bufs, acc, sid, 64)
    plsc.subcore_barrier()

    def gather_src(j):
        return hs_hbm.at[sidx_v.at[j]]

    drain_src = hs_hbm.at[pl.ds(0, _CHUNK)]
    _edge_pipeline(gather_src, drain_src, didx_v, bufs, acc, gsems, ssems,
                   _CPT)

    plsc.subcore_barrier()
    rbuf = bufs.at[0].at[pl.ds(0, 80)]
    for j in range(_RPT // 80):
        off = sid * _RPT + j * 80
        pltpu.sync_copy(acc.at[pl.ds(off, 80)], rbuf)
        pltpu.sync_copy(rbuf,
                        out_hbm.at[pl.ds(off, 80), pl.ds(cid * 64, 64)])


@functools.partial(
    pl.kernel,
    out_type=jax.ShapeDtypeStruct((_NPAD, _D), jnp.float32),
    mesh=_mesh,
    compiler_params=_cp_flat,
    scratch_types=[
        pltpu.VMEM((_CPW, _CHUNK), jnp.int32),
        pltpu.VMEM((_CPW, _CHUNK), jnp.int32),
        pltpu.VMEM((_NBUF, _CHUNK, _DP), jnp.float32),
        pltpu.VMEM_SHARED((_NPAD, _DP), jnp.float32),
    ] + [pltpu.SemaphoreType.DMA] * (2 * _NBUF),
)
def _agg48(hs_hbm, src_hbm, dst_hbm, out_hbm,
           sidx_v, didx_v, bufs, acc, *sems):
    cid = lax.axis_index("c")
    sid = lax.axis_index("s")
    wid = cid * 16 + sid
    gsems, ssems = sems[:_NBUF], sems[_NBUF:]
    pltpu.sync_copy(src_hbm.at[wid], sidx_v)
    pltpu.sync_copy(dst_hbm.at[wid], didx_v)
    _zero_acc(bufs, acc, sid, _DP)
    plsc.subcore_barrier()

    def gather_src(j):
        return hs_hbm.at[sidx_v.at[j]]

    drain_src = hs_hbm.at[pl.ds(0, _CHUNK)]
    _edge_pipeline(gather_src, drain_src, didx_v, bufs, acc, gsems, ssems,
                   _CPW)

    plsc.subcore_barrier()
    rbuf = bufs.at[0].at[pl.ds(0, 80)]
    for j in range(_RPT // 80):
        off = sid * _RPT + j * 80
        pltpu.sync_copy(acc.at[pl.ds(off, 80)], rbuf)
        pltpu.sync_copy(rbuf,
                        out_hbm.at[pl.ds(off, 80), pl.ds(cid * _DP, _DP)])


def _deg_inv_body(p_ref, o_ref):
    s = p_ref[0] + p_ref[1]
    o_ref[...] = lax.rsqrt(jnp.maximum(s, 1.0))


def _scale_body(x_ref, s_ref, o_ref):
    o_ref[...] = x_ref[...] * s_ref[...]


def _post_body(part_ref, h_ref, w_ref, l_ref, g_ref, be_ref, dini_ref,
               dino_ref, w2_ref, h1_ref, x1_ref, *, fold_w2):
    agg = part_ref[0:_N, :] * dini_ref[...]
    t = (jnp.dot(agg, w_ref[...], preferred_element_type=jnp.float32)
         + jnp.dot(h_ref[...], l_ref[...], preferred_element_type=jnp.float32))
    mu = jnp.mean(t, axis=0, keepdims=True)
    var = jnp.mean((t - mu) ** 2, axis=0, keepdims=True)
    t = g_ref[...] * (t - mu) * lax.rsqrt(var + _EPS) + be_ref[...]
    h1 = jnp.maximum(t, 0.0)
    h1_ref[...] = h1
    hs1 = h1 * dino_ref[...]
    if fold_w2:
        x1_ref[...] = jnp.dot(hs1, w2_ref[...],
                              preferred_element_type=jnp.float32)
    else:
        x1_ref[...] = hs1


def _final_body(part_ref, h_ref, l_ref, b_ref, dini_ref, o_ref):
    agg = ((part_ref[0:_N, 0:40] + part_ref[0:_N, _DP:_DP + 40])
           * dini_ref[...])
    o_ref[...] = (agg + b_ref[...]
                  + jnp.dot(h_ref[...], l_ref[...],
                            preferred_element_type=jnp.float32))


_deg_inv = pl.pallas_call(
    _deg_inv_body,
    out_shape=jax.ShapeDtypeStruct((2, 80, 128), jnp.float32))

_scale = pl.pallas_call(
    _scale_body,
    out_shape=jax.ShapeDtypeStruct((_N, _D), jnp.float32))

_post0 = pl.pallas_call(
    functools.partial(_post_body, fold_w2=False),
    out_shape=(jax.ShapeDtypeStruct((_N, _D), jnp.float32),
               jax.ShapeDtypeStruct((_N, _D), jnp.float32)))

_post1 = pl.pallas_call(
    functools.partial(_post_body, fold_w2=True),
    out_shape=(jax.ShapeDtypeStruct((_N, _D), jnp.float32),
               jax.ShapeDtypeStruct((_N, _DP), jnp.float32)))

_final = pl.pallas_call(
    _final_body,
    out_shape=jax.ShapeDtypeStruct((_N, 40), jnp.float32))


def kernel(feat, edge_index, W0, W1, W2, L0, L1, L2, b2, g0, be0, g1, be1):
    src = edge_index[0]
    dst = edge_index[1]
    src2 = jnp.concatenate([2 * src, 2 * src + 1])
    src16 = src2.reshape(32, _CPT, _CHUNK)
    dst16 = dst.reshape(16, _CPT, _CHUNK)
    src32 = src.reshape(32, _CPW, _CHUNK)
    dst32 = dst.reshape(32, _CPW, _CHUNK)
    idxrows = jnp.arange(2 * _HROWS, dtype=jnp.int32).reshape(10, 128)
    W2p = jnp.pad(W2, ((0, 0), (0, _DP - 40)))
    g0r, be0r = g0.reshape(1, _D), be0.reshape(1, _D)
    g1r, be1r = g1.reshape(1, _D), be1.reshape(1, _D)
    b2r = b2.reshape(1, 40)

    degh = _deg_kernel(src, dst, idxrows)
    dinv = _deg_inv(degh.reshape(2, 2, 80, 128))
    dcol = dinv.reshape(2, _NPAD, 1)[:, :_N, :]
    dino, dini = dcol[0], dcol[1]

    hs0 = _scale(feat, dino)
    part0 = _agg_half(hs0.reshape(2 * _N, 64), src16, dst16)
    h1, hs1 = _post0(part0, feat, W0, L0, g0r, be0r, dini, dino, W2p)
    part1 = _agg_half(hs1.reshape(2 * _N, 64), src16, dst16)
    h2, z2 = _post1(part1, h1, W1, L1, g1r, be1r, dini, dino, W2p)
    part2 = _agg48(z2, src32, dst32)
    return _final(part2, h2, L2, b2r, dini)


# speedup vs baseline: 1.0433x; 1.0433x over previous
import dataclasses
import functools

import jax
import jax.numpy as jnp
from jax import lax
from jax.experimental import pallas as pl
from jax.experimental.pallas import tpu as pltpu
from jax.experimental.pallas import tpu_sc as plsc

_N = 10000
_NPAD = 10240
_E = 320000
_D = 128
_DP = 48
_EPS = 1e-5
_CHUNK = 125
_NBUF = 5
_EPT = _E // 16
_CPT = _EPT // _CHUNK
_EPW = _E // 32
_CPW = _EPW // _CHUNK
_HROWS = _NPAD // 16
_RPT = _NPAD // 16

_mesh = plsc.VectorSubcoreMesh(core_axis_name="c", subcore_axis_name="s")

_cp = pltpu.CompilerParams()
if "needs_layout_passes" in pltpu.CompilerParams.__dataclass_fields__:
    _cp = dataclasses.replace(_cp, needs_layout_passes=False)
_cp_flat = dataclasses.replace(pltpu.CompilerParams(),
                               use_tc_tiling_on_sc=False)
_cp_flat_nl = dataclasses.replace(_cp, use_tc_tiling_on_sc=False)


@functools.partial(
    pl.kernel,
    out_type=jax.ShapeDtypeStruct((2 * 2 * _HROWS, 16), jnp.float32),
    mesh=_mesh,
    compiler_params=_cp_flat_nl,
    scratch_types=[
        pltpu.VMEM((_EPW,), jnp.int32),
        pltpu.VMEM((_EPW,), jnp.int32),
        pltpu.VMEM((10, 128), jnp.int32),
        pltpu.VMEM((_HROWS, 16), jnp.float32),
        pltpu.VMEM((_HROWS, 16), jnp.float32),
        pltpu.VMEM_SHARED((2 * _HROWS, 16), jnp.float32),
    ],
)
def _deg_kernel(src_hbm, dst_hbm, idxrows_hbm, out_hbm,
                src_v, dst_v, idxr_v, hs_v, hd_v, spm):
    cid = lax.axis_index("c")
    sid = lax.axis_index("s")
    wid = cid * 16 + sid
    base = wid * _EPW
    pltpu.sync_copy(src_hbm.at[pl.ds(base, _EPW)], src_v)
    pltpu.sync_copy(dst_hbm.at[pl.ds(base, _EPW)], dst_v)
    pltpu.sync_copy(idxrows_hbm, idxr_v)

    zero16 = jnp.zeros((16,), jnp.float32)

    @pl.loop(0, _HROWS)
    def _(r):
        hs_v[r, :] = zero16
        hd_v[r, :] = zero16

    @pl.when(sid == 0)
    def _():
        pltpu.sync_copy(hs_v, spm.at[pl.ds(0, _HROWS)])
        pltpu.sync_copy(hd_v, spm.at[pl.ds(_HROWS, _HROWS)])

    plsc.subcore_barrier()

    ones = jnp.ones((16,), jnp.float32)

    @pl.loop(0, _EPW // 16)
    def _(i):
        s = src_v[pl.ds(i * 16, 16)]
        d = dst_v[pl.ds(i * 16, 16)]
        plsc.addupdate_scatter(hs_v, [s >> 4, s & 15], ones)
        plsc.addupdate_scatter(hd_v, [d >> 4, d & 15], ones)

    for j in range(5):
        pltpu.sync_copy(hs_v.at[pl.ds(j * 128, 128)],
                        spm.at[idxr_v.at[j]], add=True)
        pltpu.sync_copy(hd_v.at[pl.ds(j * 128, 128)],
                        spm.at[idxr_v.at[j + 5]], add=True)

    plsc.subcore_barrier()

    off = sid * (2 * _HROWS // 16)
    pltpu.sync_copy(spm.at[pl.ds(off, 2 * _HROWS // 16)],
                    out_hbm.at[pl.ds(cid * 2 * _HROWS + off, 2 * _HROWS // 16)])


def _zero_acc(bufs, acc, sid, width):
    zvec = jnp.zeros((16,), jnp.float32)

    @pl.loop(0, 80)
    def _(r):
        for v in range(width // 16):
            bufs[0, r, pl.ds(v * 16, 16)] = zvec

    zsrc = bufs.at[0].at[pl.ds(0, 80)]
    for j in range(_RPT // 80):
        pltpu.sync_copy(zsrc, acc.at[pl.ds(sid * _RPT + j * 80, 80)])


def _readout(bufs, acc, out_hbm, cid, sid):
    rbuf = bufs.at[0].at[pl.ds(0, 80)]
    for j in range(_RPT // 80):
        off = sid * _RPT + j * 80
        pltpu.sync_copy(acc.at[pl.ds(off, 80)], rbuf)
        pltpu.sync_copy(rbuf, out_hbm.at[pl.ds(cid * _NPAD + off, 80)])


def _edge_pipeline(gather_src, drain_src, didx_v, bufs, acc, gsems,
                   n_chunks):
    rounds = n_chunks // _NBUF

    def buf(b):
        return bufs.at[b].at[pl.ds(0, _CHUNK)]

    for b in range(_NBUF):
        pltpu.async_copy(gather_src(b), buf(b), gsems[b])

    @pl.loop(0, rounds)
    def _(t):
        for b in range(_NBUF):
            j = t * _NBUF + b
            pltpu.make_async_copy(drain_src, buf(b), gsems[b]).wait()
            pltpu.sync_copy(buf(b), acc.at[didx_v.at[j]], add=True)

            @pl.when(t < rounds - 1)
            def _():
                pltpu.async_copy(gather_src(j + _NBUF), buf(b), gsems[b])


@functools.partial(
    pl.kernel,
    out_type=jax.ShapeDtypeStruct((_NPAD, _D), jnp.float32),
    mesh=_mesh,
    compiler_params=_cp_flat,
    scratch_types=[
        pltpu.VMEM((_CPT, _CHUNK), jnp.int32),
        pltpu.VMEM((_CPT, _CHUNK), jnp.int32),
        pltpu.VMEM((_NBUF, _CHUNK, 64), jnp.float32),
        pltpu.VMEM_SHARED((_NPAD, 64), jnp.float32),
    ] + [pltpu.SemaphoreType.DMA] * _NBUF,
)
def _agg_half(hs_hbm, src_hbm, dst_hbm, out_hbm,
              sidx_v, didx_v, bufs, acc, *gsems):
    cid = lax.axis_index("c")
    sid = lax.axis_index("s")
    pltpu.sync_copy(src_hbm.at[cid * 16 + sid], sidx_v)
    pltpu.sync_copy(dst_hbm.at[sid], didx_v)
    _zero_acc(bufs, acc, sid, 64)
    plsc.subcore_barrier()

    def gather_src(j):
        return hs_hbm.at[sidx_v.at[j]]

    drain_src = hs_hbm.at[pl.ds(0, _CHUNK)]
    _edge_pipeline(gather_src, drain_src, didx_v, bufs, acc, gsems, _CPT)

    plsc.subcore_barrier()
    rbuf = bufs.at[0].at[pl.ds(0, 80)]
    for j in range(_RPT // 80):
        off = sid * _RPT + j * 80
        pltpu.sync_copy(acc.at[pl.ds(off, 80)], rbuf)
        pltpu.sync_copy(rbuf,
                        out_hbm.at[pl.ds(off, 80), pl.ds(cid * 64, 64)])


@functools.partial(
    pl.kernel,
    out_type=jax.ShapeDtypeStruct((_NPAD, _D), jnp.float32),
    mesh=_mesh,
    compiler_params=_cp_flat,
    scratch_types=[
        pltpu.VMEM((_CPW, _CHUNK), jnp.int32),
        pltpu.VMEM((_CPW, _CHUNK), jnp.int32),
        pltpu.VMEM((_NBUF, _CHUNK, _DP), jnp.float32),
        pltpu.VMEM_SHARED((_NPAD, _DP), jnp.float32),
    ] + [pltpu.SemaphoreType.DMA] * _NBUF,
)
def _agg48(hs_hbm, src_hbm, dst_hbm, out_hbm,
           sidx_v, didx_v, bufs, acc, *gsems):
    cid = lax.axis_index("c")
    sid = lax.axis_index("s")
    wid = cid * 16 + sid
    pltpu.sync_copy(src_hbm.at[wid], sidx_v)
    pltpu.sync_copy(dst_hbm.at[wid], didx_v)
    _zero_acc(bufs, acc, sid, _DP)
    plsc.subcore_barrier()

    def gather_src(j):
        return hs_hbm.at[sidx_v.at[j]]

    drain_src = hs_hbm.at[pl.ds(0, _CHUNK)]
    _edge_pipeline(gather_src, drain_src, didx_v, bufs, acc, gsems, _CPW)

    plsc.subcore_barrier()
    rbuf = bufs.at[0].at[pl.ds(0, 80)]
    for j in range(_RPT // 80):
        off = sid * _RPT + j * 80
        pltpu.sync_copy(acc.at[pl.ds(off, 80)], rbuf)
        pltpu.sync_copy(rbuf,
                        out_hbm.at[pl.ds(off, 80), pl.ds(cid * _DP, _DP)])


def _deg_inv_body(p_ref, o_ref):
    s = p_ref[0] + p_ref[1]
    o_ref[...] = lax.rsqrt(jnp.maximum(s, 1.0))


def _scale_body(x_ref, s_ref, o_ref):
    o_ref[...] = x_ref[...] * s_ref[...]


def _post_body(part_ref, h_ref, w_ref, l_ref, g_ref, be_ref, dini_ref,
               dino_ref, w2_ref, h1_ref, x1_ref, *, fold_w2):
    agg = part_ref[0:_N, :] * dini_ref[...]
    t = (jnp.dot(agg, w_ref[...], preferred_element_type=jnp.float32)
         + jnp.dot(h_ref[...], l_ref[...], preferred_element_type=jnp.float32))
    mu = jnp.mean(t, axis=0, keepdims=True)
    var = jnp.mean((t - mu) ** 2, axis=0, keepdims=True)
    t = g_ref[...] * (t - mu) * lax.rsqrt(var + _EPS) + be_ref[...]
    h1 = jnp.maximum(t, 0.0)
    h1_ref[...] = h1
    hs1 = h1 * dino_ref[...]
    if fold_w2:
        x1_ref[...] = jnp.dot(hs1, w2_ref[...],
                              preferred_element_type=jnp.float32)
    else:
        x1_ref[...] = hs1


def _final_body(part_ref, h_ref, l_ref, b_ref, dini_ref, o_ref):
    agg = ((part_ref[0:_N, 0:40] + part_ref[0:_N, _DP:_DP + 40])
           * dini_ref[...])
    o_ref[...] = (agg + b_ref[...]
                  + jnp.dot(h_ref[...], l_ref[...],
                            preferred_element_type=jnp.float32))


_deg_inv = pl.pallas_call(
    _deg_inv_body,
    out_shape=jax.ShapeDtypeStruct((2, 80, 128), jnp.float32))

_scale = pl.pallas_call(
    _scale_body,
    out_shape=jax.ShapeDtypeStruct((_N, _D), jnp.float32))

_post0 = pl.pallas_call(
    functools.partial(_post_body, fold_w2=False),
    out_shape=(jax.ShapeDtypeStruct((_N, _D), jnp.float32),
               jax.ShapeDtypeStruct((_N, _D), jnp.float32)))

_post1 = pl.pallas_call(
    functools.partial(_post_body, fold_w2=True),
    out_shape=(jax.ShapeDtypeStruct((_N, _D), jnp.float32),
               jax.ShapeDtypeStruct((_N, _DP), jnp.float32)))

_final = pl.pallas_call(
    _final_body,
    out_shape=jax.ShapeDtypeStruct((_N, 40), jnp.float32))


def kernel(feat, edge_index, W0, W1, W2, L0, L1, L2, b2, g0, be0, g1, be1):
    src = edge_index[0]
    dst = edge_index[1]
    src2 = jnp.concatenate([2 * src, 2 * src + 1])
    src16 = src2.reshape(32, _CPT, _CHUNK)
    dst16 = dst.reshape(16, _CPT, _CHUNK)
    src32 = src.reshape(32, _CPW, _CHUNK)
    dst32 = dst.reshape(32, _CPW, _CHUNK)
    idxrows = jnp.arange(2 * _HROWS, dtype=jnp.int32).reshape(10, 128)
    W2p = jnp.pad(W2, ((0, 0), (0, _DP - 40)))
    g0r, be0r = g0.reshape(1, _D), be0.reshape(1, _D)
    g1r, be1r = g1.reshape(1, _D), be1.reshape(1, _D)
    b2r = b2.reshape(1, 40)

    degh = _deg_kernel(src, dst, idxrows)
    dinv = _deg_inv(degh.reshape(2, 2, 80, 128))
    dcol = dinv.reshape(2, _NPAD, 1)[:, :_N, :]
    dino, dini = dcol[0], dcol[1]

    hs0 = _scale(feat, dino)
    part0 = _agg_half(hs0.reshape(2 * _N, 64), src16, dst16)
    h1, hs1 = _post0(part0, feat, W0, L0, g0r, be0r, dini, dino, W2p)
    part1 = _agg_half(hs1.reshape(2 * _N, 64), src16, dst16)
    h2, z2 = _post1(part1, h1, W1, L1, g1r, be1r, dini, dino, W2p)
    part2 = _agg48(z2, src32, dst32)
    return _final(part2, h2, L2, b2r, dini)
